# baseline (speedup 1.0000x reference)
import jax
import jax.numpy as jnp
from jax import lax
from jax.experimental import pallas as pl
from jax.experimental.pallas import tpu as pltpu
from jax.experimental.pallas import tpu_sc as plsc

NEGATIVE_RATIO = 3.0
EPS = 1e-06
NEGATIVE_OVERRIDE = 0.1

NC, NS, L = 1, 16, 16
NW = NC * NS
N_PIX = 1280 * 1280
PER_W = N_PIX // NW
CHUNK = 12800
N_CHUNKS = PER_W // CHUNK
UNROLL = 4
NB = 2048
LOSS_HI = 9.25
BIN_SCALE = NB / LOSS_HI

LN2 = 0.6931471805599453
SQRT2 = 1.4142135623730951
_P5 = (-0.14338312557646554, 0.22070299637323865, -0.2539783176623159,
       0.33256777805214044, -0.4999036158820021, 1.0000046848976198)


def _ln(x):
    xi = lax.bitcast_convert_type(x, jnp.int32)
    e = (xi >> 23) & 0xFF
    m = lax.bitcast_convert_type((xi & 0x007FFFFF) | 0x3F800000, jnp.float32)
    big = m > SQRT2
    m = jnp.where(big, m * 0.5, m)
    ef = (e + jnp.where(big, -126, -127)).astype(jnp.float32)
    r = m - 1.0
    p = _P5[0]
    for c in _P5[1:]:
        p = p * r + c
    return ef * LN2 + r * p


def _losses(p, g, m, ign):
    vm = jnp.where(g == ign, 0.0, m)
    x = jnp.where(g == 1.0, p, 1.0 - p)
    loss = -jnp.maximum(_ln(x), -100.0)
    posf = vm * g
    negf = vm - posf
    return loss, posf, negf


def _sc_stats_body(pred_h, gt_h, mask_h, ign_h, part_h,
                   pred_v, gt_v, mask_v, ign_v, part_v):
    wid = lax.axis_index("s") * NC + lax.axis_index("c")
    base = wid * PER_W

    pltpu.sync_copy(ign_h, ign_v)
    zeros = jnp.zeros((L,), jnp.float32)
    ign = ign_v[...]
    acc = ((zeros, zeros, zeros, zeros),) * UNROLL

    def lane_stats(sl, a):
        ps, pc, ncnt, nsum = a
        loss, posf, negf = _losses(pred_v[sl], gt_v[sl], mask_v[sl], ign)
        return (ps + loss * posf, pc + posf, ncnt + negf, nsum + loss * negf)

    def vec_body(j, accs):
        base_sl = pl.multiple_of(j * (L * UNROLL), L * UNROLL)
        return tuple(
            lane_stats(pl.ds(base_sl + u * L, L), accs[u])
            for u in range(UNROLL)
        )

    for ci in range(N_CHUNKS):
        off = pl.multiple_of(base + ci * CHUNK, CHUNK)
        pltpu.sync_copy(pred_h.at[pl.ds(off, CHUNK)], pred_v)
        pltpu.sync_copy(gt_h.at[pl.ds(off, CHUNK)], gt_v)
        pltpu.sync_copy(mask_h.at[pl.ds(off, CHUNK)], mask_v)
        acc = lax.fori_loop(0, CHUNK // (L * UNROLL), vec_body, acc)

    ps, pc, ncnt, nsum = acc[0]
    for u in range(1, UNROLL):
        ps = ps + acc[u][0]
        pc = pc + acc[u][1]
        ncnt = ncnt + acc[u][2]
        nsum = nsum + acc[u][3]
    part_v[0, :] = ps
    part_v[1, :] = pc
    part_v[2, :] = ncnt
    part_v[3, :] = nsum
    pltpu.sync_copy(part_v, part_h.at[wid])


def _sc_hist_body(pred_h, gt_h, mask_h, ign_h, hist_h,
                  pred_v, gt_v, mask_v, ign_v, hcnt_v, hsum_v):
    wid = lax.axis_index("s") * NC + lax.axis_index("c")
    base = wid * PER_W

    pltpu.sync_copy(ign_h, ign_v)
    zeros = jnp.zeros((L,), jnp.float32)
    for j in range(NB // L):
        hcnt_v[j * L:(j + 1) * L] = zeros
        hsum_v[j * L:(j + 1) * L] = zeros

    ign = ign_v[...]
    ones = jnp.ones((L,), jnp.float32)

    def vec_body(j, carry):
        sl = pl.ds(pl.multiple_of(j * L, L), L)
        loss, _, negf = _losses(pred_v[sl], gt_v[sl], mask_v[sl], ign)
        nl = loss * negf
        bi = jnp.minimum((nl * BIN_SCALE).astype(jnp.int32), NB - 1)
        sel = negf > 0.0
        plsc.addupdate_scatter(hcnt_v, [bi], ones, mask=sel)
        plsc.addupdate_scatter(hsum_v, [bi], nl, mask=sel)
        return carry

    for ci in range(N_CHUNKS):
        off = pl.multiple_of(base + ci * CHUNK, CHUNK)
        pltpu.sync_copy(pred_h.at[pl.ds(off, CHUNK)], pred_v)
        pltpu.sync_copy(gt_h.at[pl.ds(off, CHUNK)], gt_v)
        pltpu.sync_copy(mask_h.at[pl.ds(off, CHUNK)], mask_v)
        lax.fori_loop(0, CHUNK // L, vec_body, 0)

    pltpu.sync_copy(hcnt_v, hist_h.at[wid, 0])
    pltpu.sync_copy(hsum_v, hist_h.at[wid, 1])


def kernel(pred, gt, mask, ignore_label=255):
    predf = pred.reshape(-1)
    gtf = gt.reshape(-1)
    maskf = mask.reshape(-1)
    ignf = jnp.full((L,), ignore_label, jnp.float32)

    mesh = plsc.VectorSubcoreMesh(core_axis_name="c", subcore_axis_name="s",
                                  num_cores=NC)
    params = pltpu.CompilerParams(needs_layout_passes=False)

    part = pl.kernel(
        _sc_stats_body,
        out_type=jax.ShapeDtypeStruct((NW, 4, L), jnp.float32),
        mesh=mesh,
        compiler_params=params,
        scratch_types=[
            pltpu.VMEM((CHUNK,), jnp.float32),
            pltpu.VMEM((CHUNK,), jnp.float32),
            pltpu.VMEM((CHUNK,), jnp.float32),
            pltpu.VMEM((L,), jnp.float32),
            pltpu.VMEM((4, L), jnp.float32),
        ],
    )(predf, gtf, maskf, ignf)

    pos_sum = part[:, 0, :].sum()
    pos_cnt = jnp.floor(part[:, 1, :].sum())
    neg_tot = jnp.floor(part[:, 2, :].sum())
    neg_sum = part[:, 3, :].sum()

    k = jnp.where(
        pos_cnt >= 1.0,
        jnp.minimum(neg_tot, jnp.floor(pos_cnt * NEGATIVE_RATIO)),
        jnp.floor(neg_tot * NEGATIVE_OVERRIDE),
    )

    def topk_from_hist():
        hist = pl.kernel(
            _sc_hist_body,
            out_type=jax.ShapeDtypeStruct((NW, 2, NB), jnp.float32),
            mesh=mesh,
            compiler_params=params,
            scratch_types=[
                pltpu.VMEM((CHUNK,), jnp.float32),
                pltpu.VMEM((CHUNK,), jnp.float32),
                pltpu.VMEM((CHUNK,), jnp.float32),
                pltpu.VMEM((L,), jnp.float32),
                pltpu.VMEM((NB,), jnp.float32),
                pltpu.VMEM((NB,), jnp.float32),
            ],
        )(predf, gtf, maskf, ignf)
        hcnt = hist[:, 0, :].sum(axis=0)
        hsum = hist[:, 1, :].sum(axis=0)
        cc = jnp.cumsum(hcnt[::-1])[::-1]
        cs = jnp.cumsum(hsum[::-1])[::-1]
        cc1 = jnp.concatenate([cc, jnp.zeros((1,), jnp.float32)])
        cs1 = jnp.concatenate([cs, jnp.zeros((1,), jnp.float32)])
        t = jnp.clip((cc >= k).sum() - 1, 0, NB - 1)
        frac = (k - cc1[t + 1]) / jnp.maximum(hcnt[t], 1.0)
        return jnp.where(k >= 1.0, cs1[t + 1] + frac * hsum[t], 0.0)

    neg_top = lax.cond(k >= neg_tot, lambda: neg_sum, topk_from_hist)
    return (pos_sum + neg_top) / (pos_cnt + k + EPS)


# speedup vs baseline: 1.4861x; 1.4861x over previous
import jax
import jax.numpy as jnp
from jax import lax
from jax.experimental import pallas as pl
from jax.experimental.pallas import tpu as pltpu
from jax.experimental.pallas import tpu_sc as plsc

NEGATIVE_RATIO = 3.0
EPS = 1e-06
NEGATIVE_OVERRIDE = 0.1

NC, NS, L = 2, 16, 16
NW = NC * NS
H, W = 1280, 1280
N_PIX = H * W
TR = 8
N_BANDS = H // TR
BANDS_W = N_BANDS // NW
CHUNK = TR * W
UNROLL = 4
NB = 2048
LOSS_HI = 9.25
BIN_SCALE = NB / LOSS_HI

LN2 = 0.6931471805599453
SQRT2 = 1.4142135623730951
_P5 = (-0.14338312557646554, 0.22070299637323865, -0.2539783176623159,
       0.33256777805214044, -0.4999036158820021, 1.0000046848976198)


def _ln(x):
    xi = lax.bitcast_convert_type(x, jnp.int32)
    e = (xi >> 23) & 0xFF
    m = lax.bitcast_convert_type((xi & 0x007FFFFF) | 0x3F800000, jnp.float32)
    big = m > SQRT2
    m = jnp.where(big, m * 0.5, m)
    ef = (e + jnp.where(big, -126, -127)).astype(jnp.float32)
    r = m - 1.0
    p = _P5[0]
    for c in _P5[1:]:
        p = p * r + c
    return ef * LN2 + r * p


def _losses(p, g, m, ign):
    vm = jnp.where(g == ign, 0.0, m)
    x = jnp.where(g == 1.0, p, 1.0 - p)
    loss = -jnp.maximum(_ln(x), -100.0)
    posf = vm * g
    negf = vm - posf
    return loss, posf, negf


def _sc_stats_body(pred_h, gt_h, mask_h, ign_h, part_h,
                   pred_v, gt_v, mask_v, ign_v, part_v):
    wid = lax.axis_index("s") * NC + lax.axis_index("c")

    pltpu.sync_copy(ign_h, ign_v)
    zeros = jnp.zeros((L,), jnp.float32)
    ign = ign_v[...]
    acc = ((zeros, zeros, zeros, zeros),) * UNROLL

    def lane_stats(row, sl, a):
        ps, pc, ncnt, nsum = a
        loss, posf, negf = _losses(
            pred_v[row, sl], gt_v[row, sl], mask_v[row, sl], ign)
        return (ps + loss * posf, pc + posf, ncnt + negf, nsum + loss * negf)

    def make_col_body(row):
        def col_body(j, accs):
            col = pl.multiple_of(j * (L * UNROLL), L * UNROLL)
            return tuple(
                lane_stats(row, pl.ds(col + u * L, L), accs[u])
                for u in range(UNROLL)
            )
        return col_body

    for ci in range(BANDS_W):
        r0 = pl.multiple_of((wid * BANDS_W + ci) * TR, TR)
        pltpu.sync_copy(pred_h.at[pl.ds(r0, TR), :], pred_v)
        pltpu.sync_copy(gt_h.at[pl.ds(r0, TR), :], gt_v)
        pltpu.sync_copy(mask_h.at[pl.ds(r0, TR), :], mask_v)
        for row in range(TR):
            acc = lax.fori_loop(0, W // (L * UNROLL), make_col_body(row), acc)

    ps, pc, ncnt, nsum = acc[0]
    for u in range(1, UNROLL):
        ps = ps + acc[u][0]
        pc = pc + acc[u][1]
        ncnt = ncnt + acc[u][2]
        nsum = nsum + acc[u][3]
    part_v[0, :] = ps
    part_v[1, :] = pc
    part_v[2, :] = ncnt
    part_v[3, :] = nsum
    pltpu.sync_copy(part_v, part_h.at[wid])


def _sc_hist_body(pred_h, gt_h, mask_h, ign_h, hist_h,
                  pred_v, gt_v, mask_v, ign_v, hcnt_v, hsum_v):
    wid = lax.axis_index("s") * NC + lax.axis_index("c")

    pltpu.sync_copy(ign_h, ign_v)
    zeros = jnp.zeros((L,), jnp.float32)
    for j in range(NB // L):
        hcnt_v[j * L:(j + 1) * L] = zeros
        hsum_v[j * L:(j + 1) * L] = zeros

    ign = ign_v[...]
    ones = jnp.ones((L,), jnp.float32)

    def make_col_body(row):
        def col_body(j, carry):
            sl = pl.ds(pl.multiple_of(j * L, L), L)
            loss, _, negf = _losses(
                pred_v[row, sl], gt_v[row, sl], mask_v[row, sl], ign)
            nl = loss * negf
            bi = jnp.minimum((nl * BIN_SCALE).astype(jnp.int32), NB - 1)
            sel = negf > 0.0
            plsc.addupdate_scatter(hcnt_v, [bi], ones, mask=sel)
            plsc.addupdate_scatter(hsum_v, [bi], nl, mask=sel)
            return carry
        return col_body

    for ci in range(BANDS_W):
        r0 = pl.multiple_of((wid * BANDS_W + ci) * TR, TR)
        pltpu.sync_copy(pred_h.at[pl.ds(r0, TR), :], pred_v)
        pltpu.sync_copy(gt_h.at[pl.ds(r0, TR), :], gt_v)
        pltpu.sync_copy(mask_h.at[pl.ds(r0, TR), :], mask_v)
        for row in range(TR):
            lax.fori_loop(0, W // L, make_col_body(row), 0)

    pltpu.sync_copy(hcnt_v, hist_h.at[wid, 0])
    pltpu.sync_copy(hsum_v, hist_h.at[wid, 1])


def kernel(pred, gt, mask, ignore_label=255):
    predf = pred.reshape(H, W)
    gtf = gt.reshape(H, W)
    maskf = mask.reshape(H, W)
    ignf = jnp.full((L,), ignore_label, jnp.float32)

    mesh = plsc.VectorSubcoreMesh(core_axis_name="c", subcore_axis_name="s")
    params = pltpu.CompilerParams(needs_layout_passes=False,
                                  use_tc_tiling_on_sc=True)

    part = pl.kernel(
        _sc_stats_body,
        out_type=jax.ShapeDtypeStruct((NW, 4, L), jnp.float32),
        mesh=mesh,
        compiler_params=params,
        scratch_types=[
            pltpu.VMEM((TR, W), jnp.float32),
            pltpu.VMEM((TR, W), jnp.float32),
            pltpu.VMEM((TR, W), jnp.float32),
            pltpu.VMEM((L,), jnp.float32),
            pltpu.VMEM((4, L), jnp.float32),
        ],
    )(predf, gtf, maskf, ignf)

    pos_sum = part[:, 0, :].sum()
    pos_cnt = jnp.floor(part[:, 1, :].sum())
    neg_tot = jnp.floor(part[:, 2, :].sum())
    neg_sum = part[:, 3, :].sum()

    k = jnp.where(
        pos_cnt >= 1.0,
        jnp.minimum(neg_tot, jnp.floor(pos_cnt * NEGATIVE_RATIO)),
        jnp.floor(neg_tot * NEGATIVE_OVERRIDE),
    )

    def topk_from_hist():
        hist = pl.kernel(
            _sc_hist_body,
            out_type=jax.ShapeDtypeStruct((NW, 2, NB), jnp.float32),
            mesh=mesh,
            compiler_params=params,
            scratch_types=[
                pltpu.VMEM((TR, W), jnp.float32),
                pltpu.VMEM((TR, W), jnp.float32),
                pltpu.VMEM((TR, W), jnp.float32),
                pltpu.VMEM((L,), jnp.float32),
                pltpu.VMEM((NB,), jnp.float32),
                pltpu.VMEM((NB,), jnp.float32),
            ],
        )(predf, gtf, maskf, ignf)
        hcnt = hist[:, 0, :].sum(axis=0)
        hsum = hist[:, 1, :].sum(axis=0)
        cc = jnp.cumsum(hcnt[::-1])[::-1]
        cs = jnp.cumsum(hsum[::-1])[::-1]
        cc1 = jnp.concatenate([cc, jnp.zeros((1,), jnp.float32)])
        cs1 = jnp.concatenate([cs, jnp.zeros((1,), jnp.float32)])
        t = jnp.clip((cc >= k).sum() - 1, 0, NB - 1)
        frac = (k - cc1[t + 1]) / jnp.maximum(hcnt[t], 1.0)
        return jnp.where(k >= 1.0, cs1[t + 1] + frac * hsum[t], 0.0)

    neg_top = lax.cond(k >= neg_tot, lambda: neg_sum, topk_from_hist)
    return (pos_sum + neg_top) / (pos_cnt + k + EPS)


# speedup vs baseline: 1.6346x; 1.0999x over previous
import jax
import jax.numpy as jnp
from jax import lax
from jax.experimental import pallas as pl
from jax.experimental.pallas import tpu as pltpu
from jax.experimental.pallas import tpu_sc as plsc

NEGATIVE_RATIO = 3.0
EPS = 1e-06
NEGATIVE_OVERRIDE = 0.1

NC, NS, L = 2, 16, 16
NW = NC * NS
H, W = 1280, 1280
N_PIX = H * W
TR = 8
N_BANDS = H // TR
BANDS_W = N_BANDS // NW
CHUNK = TR * W
UNROLL = 4
NB = 2048
LOSS_HI = 9.25
BIN_SCALE = NB / LOSS_HI

LN2 = 0.6931471805599453
SQRT2 = 1.4142135623730951
_P5 = (-0.14338312557646554, 0.22070299637323865, -0.2539783176623159,
       0.33256777805214044, -0.4999036158820021, 1.0000046848976198)


def _ln(x):
    xi = lax.bitcast_convert_type(x, jnp.int32)
    e = (xi >> 23) & 0xFF
    m = lax.bitcast_convert_type((xi & 0x007FFFFF) | 0x3F800000, jnp.float32)
    big = m > SQRT2
    m = jnp.where(big, m * 0.5, m)
    ef = (e + jnp.where(big, -126, -127)).astype(jnp.float32)
    r = m - 1.0
    p = _P5[0]
    for c in _P5[1:]:
        p = p * r + c
    return ef * LN2 + r * p


def _losses(p, g, m, ign):
    vm = jnp.where(g == ign, 0.0, m)
    x = jnp.where(g == 1.0, p, 1.0 - p)
    loss = -jnp.maximum(_ln(x), -100.0)
    posf = vm * g
    negf = vm - posf
    return loss, posf, negf


def _sc_stats_body(pred_h, gt_h, mask_h, ign_h, part_h,
                   pred_v, gt_v, mask_v, ign_v, part_v):
    wid = lax.axis_index("s") * NC + lax.axis_index("c")

    pltpu.sync_copy(ign_h, ign_v)
    zeros = jnp.zeros((L,), jnp.float32)
    ign = ign_v[...]
    acc = ((zeros, zeros, zeros, zeros),) * UNROLL

    def lane_stats(row, sl, a):
        ps, pc, ncnt, nsum = a
        loss, posf, negf = _losses(
            pred_v[row, sl], gt_v[row, sl], mask_v[row, sl], ign)
        return (ps + loss * posf, pc + posf, ncnt + negf, nsum + loss * negf)

    def col_body(j, accs):
        col = pl.multiple_of(j * (L * UNROLL), L * UNROLL)
        accs = list(accs)
        for row in range(TR):
            for u in range(UNROLL):
                accs[u] = lane_stats(row, pl.ds(col + u * L, L), accs[u])
        return tuple(accs)

    for ci in range(BANDS_W):
        r0 = pl.multiple_of((wid * BANDS_W + ci) * TR, TR)
        pltpu.sync_copy(pred_h.at[pl.ds(r0, TR), :], pred_v)
        pltpu.sync_copy(gt_h.at[pl.ds(r0, TR), :], gt_v)
        pltpu.sync_copy(mask_h.at[pl.ds(r0, TR), :], mask_v)
        acc = lax.fori_loop(0, W // (L * UNROLL), col_body, acc)

    ps, pc, ncnt, nsum = acc[0]
    for u in range(1, UNROLL):
        ps = ps + acc[u][0]
        pc = pc + acc[u][1]
        ncnt = ncnt + acc[u][2]
        nsum = nsum + acc[u][3]
    part_v[0, :] = ps
    part_v[1, :] = pc
    part_v[2, :] = ncnt
    part_v[3, :] = nsum
    pltpu.sync_copy(part_v, part_h.at[wid])


def _sc_hist_body(pred_h, gt_h, mask_h, ign_h, hist_h,
                  pred_v, gt_v, mask_v, ign_v, hcnt_v, hsum_v):
    wid = lax.axis_index("s") * NC + lax.axis_index("c")

    pltpu.sync_copy(ign_h, ign_v)
    zeros = jnp.zeros((L,), jnp.float32)
    for j in range(NB // L):
        hcnt_v[j * L:(j + 1) * L] = zeros
        hsum_v[j * L:(j + 1) * L] = zeros

    ign = ign_v[...]
    ones = jnp.ones((L,), jnp.float32)

    def make_col_body(row):
        def col_body(j, carry):
            sl = pl.ds(pl.multiple_of(j * L, L), L)
            loss, _, negf = _losses(
                pred_v[row, sl], gt_v[row, sl], mask_v[row, sl], ign)
            nl = loss * negf
            bi = jnp.minimum((nl * BIN_SCALE).astype(jnp.int32), NB - 1)
            sel = negf > 0.0
            plsc.addupdate_scatter(hcnt_v, [bi], ones, mask=sel)
            plsc.addupdate_scatter(hsum_v, [bi], nl, mask=sel)
            return carry
        return col_body

    for ci in range(BANDS_W):
        r0 = pl.multiple_of((wid * BANDS_W + ci) * TR, TR)
        pltpu.sync_copy(pred_h.at[pl.ds(r0, TR), :], pred_v)
        pltpu.sync_copy(gt_h.at[pl.ds(r0, TR), :], gt_v)
        pltpu.sync_copy(mask_h.at[pl.ds(r0, TR), :], mask_v)
        for row in range(TR):
            lax.fori_loop(0, W // L, make_col_body(row), 0)

    pltpu.sync_copy(hcnt_v, hist_h.at[wid, 0])
    pltpu.sync_copy(hsum_v, hist_h.at[wid, 1])


def kernel(pred, gt, mask, ignore_label=255):
    predf = pred.reshape(H, W)
    gtf = gt.reshape(H, W)
    maskf = mask.reshape(H, W)
    ignf = jnp.full((L,), ignore_label, jnp.float32)

    mesh = plsc.VectorSubcoreMesh(core_axis_name="c", subcore_axis_name="s")
    params = pltpu.CompilerParams(needs_layout_passes=False,
                                  use_tc_tiling_on_sc=True)

    part = pl.kernel(
        _sc_stats_body,
        out_type=jax.ShapeDtypeStruct((NW, 4, L), jnp.float32),
        mesh=mesh,
        compiler_params=params,
        scratch_types=[
            pltpu.VMEM((TR, W), jnp.float32),
            pltpu.VMEM((TR, W), jnp.float32),
            pltpu.VMEM((TR, W), jnp.float32),
            pltpu.VMEM((L,), jnp.float32),
            pltpu.VMEM((4, L), jnp.float32),
        ],
    )(predf, gtf, maskf, ignf)

    pos_sum = part[:, 0, :].sum()
    pos_cnt = jnp.floor(part[:, 1, :].sum())
    neg_tot = jnp.floor(part[:, 2, :].sum())
    neg_sum = part[:, 3, :].sum()

    k = jnp.where(
        pos_cnt >= 1.0,
        jnp.minimum(neg_tot, jnp.floor(pos_cnt * NEGATIVE_RATIO)),
        jnp.floor(neg_tot * NEGATIVE_OVERRIDE),
    )

    def topk_from_hist():
        hist = pl.kernel(
            _sc_hist_body,
            out_type=jax.ShapeDtypeStruct((NW, 2, NB), jnp.float32),
            mesh=mesh,
            compiler_params=params,
            scratch_types=[
                pltpu.VMEM((TR, W), jnp.float32),
                pltpu.VMEM((TR, W), jnp.float32),
                pltpu.VMEM((TR, W), jnp.float32),
                pltpu.VMEM((L,), jnp.float32),
                pltpu.VMEM((NB,), jnp.float32),
                pltpu.VMEM((NB,), jnp.float32),
            ],
        )(predf, gtf, maskf, ignf)
        hcnt = hist[:, 0, :].sum(axis=0)
        hsum = hist[:, 1, :].sum(axis=0)
        cc = jnp.cumsum(hcnt[::-1])[::-1]
        cs = jnp.cumsum(hsum[::-1])[::-1]
        cc1 = jnp.concatenate([cc, jnp.zeros((1,), jnp.float32)])
        cs1 = jnp.concatenate([cs, jnp.zeros((1,), jnp.float32)])
        t = jnp.clip((cc >= k).sum() - 1, 0, NB - 1)
        frac = (k - cc1[t + 1]) / jnp.maximum(hcnt[t], 1.0)
        return jnp.where(k >= 1.0, cs1[t + 1] + frac * hsum[t], 0.0)

    neg_top = lax.cond(k >= neg_tot, lambda: neg_sum, topk_from_hist)
    return (pos_sum + neg_top) / (pos_cnt + k + EPS)


# speedup vs baseline: 1.9471x; 1.1912x over previous
import jax
import jax.numpy as jnp
from jax import lax
from jax.experimental import pallas as pl
from jax.experimental.pallas import tpu as pltpu
from jax.experimental.pallas import tpu_sc as plsc

NEGATIVE_RATIO = 3.0
EPS = 1e-06
NEGATIVE_OVERRIDE = 0.1

NC, NS, L = 2, 16, 16
NW = NC * NS
H, W = 1280, 1280
N_PIX = H * W
TR = 8
N_BANDS = H // TR
BANDS_W = N_BANDS // NW
CHUNK = TR * W
UNROLL = 4
NB = 2048
LOSS_HI = 9.25
BIN_SCALE = NB / LOSS_HI

LN2 = 0.6931471805599453
SQRT2 = 1.4142135623730951
_P5 = (-0.14338312557646554, 0.22070299637323865, -0.2539783176623159,
       0.33256777805214044, -0.4999036158820021, 1.0000046848976198)


def _ln(x):
    xi = lax.bitcast_convert_type(x, jnp.int32)
    e = (xi >> 23) & 0xFF
    m = lax.bitcast_convert_type((xi & 0x007FFFFF) | 0x3F800000, jnp.float32)
    big = m > SQRT2
    m = jnp.where(big, m * 0.5, m)
    ef = (e + jnp.where(big, -126, -127)).astype(jnp.float32)
    r = m - 1.0
    p = _P5[0]
    for c in _P5[1:]:
        p = p * r + c
    return ef * LN2 + r * p


def _losses(p, g, m, ign):
    vm = jnp.where(g == ign, 0.0, m)
    x = jnp.where(g == 1.0, p, 1.0 - p)
    loss = -jnp.maximum(_ln(x), -100.0)
    posf = vm * g
    negf = vm - posf
    return loss, posf, negf


def _sc_stats_body(pred_h, gt_h, mask_h, ign_h, part_h,
                   pred_v, gt_v, mask_v, ign_v, part_v, sem0, sem1):
    wid = lax.axis_index("s") * NC + lax.axis_index("c")
    sems = (sem0, sem1)

    pltpu.sync_copy(ign_h, ign_v)
    zeros = jnp.zeros((L,), jnp.float32)
    ign = ign_v[...]
    acc = ((zeros, zeros, zeros, zeros),) * UNROLL

    def start_band(ci):
        b = ci % 2
        r0 = pl.multiple_of((wid * BANDS_W + ci) * TR, TR)
        return [
            pltpu.async_copy(pred_h.at[pl.ds(r0, TR), :], pred_v.at[b], sems[b]),
            pltpu.async_copy(gt_h.at[pl.ds(r0, TR), :], gt_v.at[b], sems[b]),
            pltpu.async_copy(mask_h.at[pl.ds(r0, TR), :], mask_v.at[b], sems[b]),
        ]

    def make_col_body(b):
        def lane_stats(row, sl, a):
            ps, pc, ncnt, nsum = a
            loss, posf, negf = _losses(
                pred_v[b, row, sl], gt_v[b, row, sl], mask_v[b, row, sl], ign)
            return (ps + loss * posf, pc + posf,
                    ncnt + negf, nsum + loss * negf)

        def col_body(j, accs):
            col = pl.multiple_of(j * (L * UNROLL), L * UNROLL)
            accs = list(accs)
            for row in range(TR):
                for u in range(UNROLL):
                    accs[u] = lane_stats(row, pl.ds(col + u * L, L), accs[u])
            return tuple(accs)
        return col_body

    descs = start_band(0)
    for ci in range(BANDS_W):
        nxt = start_band(ci + 1) if ci + 1 < BANDS_W else []
        for d in descs:
            d.wait()
        descs = nxt
        acc = lax.fori_loop(0, W // (L * UNROLL), make_col_body(ci % 2), acc)

    ps, pc, ncnt, nsum = acc[0]
    for u in range(1, UNROLL):
        ps = ps + acc[u][0]
        pc = pc + acc[u][1]
        ncnt = ncnt + acc[u][2]
        nsum = nsum + acc[u][3]
    part_v[0, :] = ps
    part_v[1, :] = pc
    part_v[2, :] = ncnt
    part_v[3, :] = nsum
    pltpu.sync_copy(part_v, part_h.at[wid])


def _sc_hist_body(pred_h, gt_h, mask_h, ign_h, hist_h,
                  pred_v, gt_v, mask_v, ign_v, hcnt_v, hsum_v):
    wid = lax.axis_index("s") * NC + lax.axis_index("c")

    pltpu.sync_copy(ign_h, ign_v)
    zeros = jnp.zeros((L,), jnp.float32)
    for j in range(NB // L):
        hcnt_v[j * L:(j + 1) * L] = zeros
        hsum_v[j * L:(j + 1) * L] = zeros

    ign = ign_v[...]
    ones = jnp.ones((L,), jnp.float32)

    def make_col_body(row):
        def col_body(j, carry):
            sl = pl.ds(pl.multiple_of(j * L, L), L)
            loss, _, negf = _losses(
                pred_v[row, sl], gt_v[row, sl], mask_v[row, sl], ign)
            nl = loss * negf
            bi = jnp.minimum((nl * BIN_SCALE).astype(jnp.int32), NB - 1)
            sel = negf > 0.0
            plsc.addupdate_scatter(hcnt_v, [bi], ones, mask=sel)
            plsc.addupdate_scatter(hsum_v, [bi], nl, mask=sel)
            return carry
        return col_body

    for ci in range(BANDS_W):
        r0 = pl.multiple_of((wid * BANDS_W + ci) * TR, TR)
        pltpu.sync_copy(pred_h.at[pl.ds(r0, TR), :], pred_v)
        pltpu.sync_copy(gt_h.at[pl.ds(r0, TR), :], gt_v)
        pltpu.sync_copy(mask_h.at[pl.ds(r0, TR), :], mask_v)
        for row in range(TR):
            lax.fori_loop(0, W // L, make_col_body(row), 0)

    pltpu.sync_copy(hcnt_v, hist_h.at[wid, 0])
    pltpu.sync_copy(hsum_v, hist_h.at[wid, 1])


def kernel(pred, gt, mask, ignore_label=255):
    predf = pred.reshape(H, W)
    gtf = gt.reshape(H, W)
    maskf = mask.reshape(H, W)
    ignf = jnp.full((L,), ignore_label, jnp.float32)

    mesh = plsc.VectorSubcoreMesh(core_axis_name="c", subcore_axis_name="s")
    params = pltpu.CompilerParams(needs_layout_passes=False,
                                  use_tc_tiling_on_sc=True)

    part = pl.kernel(
        _sc_stats_body,
        out_type=jax.ShapeDtypeStruct((NW, 4, L), jnp.float32),
        mesh=mesh,
        compiler_params=params,
        scratch_types=[
            pltpu.VMEM((2, TR, W), jnp.float32),
            pltpu.VMEM((2, TR, W), jnp.float32),
            pltpu.VMEM((2, TR, W), jnp.float32),
            pltpu.VMEM((L,), jnp.float32),
            pltpu.VMEM((4, L), jnp.float32),
            pltpu.SemaphoreType.DMA,
            pltpu.SemaphoreType.DMA,
        ],
    )(predf, gtf, maskf, ignf)

    pos_sum = part[:, 0, :].sum()
    pos_cnt = jnp.floor(part[:, 1, :].sum())
    neg_tot = jnp.floor(part[:, 2, :].sum())
    neg_sum = part[:, 3, :].sum()

    k = jnp.where(
        pos_cnt >= 1.0,
        jnp.minimum(neg_tot, jnp.floor(pos_cnt * NEGATIVE_RATIO)),
        jnp.floor(neg_tot * NEGATIVE_OVERRIDE),
    )

    def topk_from_hist():
        hist = pl.kernel(
            _sc_hist_body,
            out_type=jax.ShapeDtypeStruct((NW, 2, NB), jnp.float32),
            mesh=mesh,
            compiler_params=params,
            scratch_types=[
                pltpu.VMEM((TR, W), jnp.float32),
                pltpu.VMEM((TR, W), jnp.float32),
                pltpu.VMEM((TR, W), jnp.float32),
                pltpu.VMEM((L,), jnp.float32),
                pltpu.VMEM((NB,), jnp.float32),
                pltpu.VMEM((NB,), jnp.float32),
            ],
        )(predf, gtf, maskf, ignf)
        hcnt = hist[:, 0, :].sum(axis=0)
        hsum = hist[:, 1, :].sum(axis=0)
        cc = jnp.cumsum(hcnt[::-1])[::-1]
        cs = jnp.cumsum(hsum[::-1])[::-1]
        cc1 = jnp.concatenate([cc, jnp.zeros((1,), jnp.float32)])
        cs1 = jnp.concatenate([cs, jnp.zeros((1,), jnp.float32)])
        t = jnp.clip((cc >= k).sum() - 1, 0, NB - 1)
        frac = (k - cc1[t + 1]) / jnp.maximum(hcnt[t], 1.0)
        return jnp.where(k >= 1.0, cs1[t + 1] + frac * hsum[t], 0.0)

    neg_top = lax.cond(k >= neg_tot, lambda: neg_sum, topk_from_hist)
    return (pos_sum + neg_top) / (pos_cnt + k + EPS)


# speedup vs baseline: 2.1150x; 1.0862x over previous
import jax
import jax.numpy as jnp
from jax import lax
from jax.experimental import pallas as pl
from jax.experimental.pallas import tpu as pltpu
from jax.experimental.pallas import tpu_sc as plsc

NEGATIVE_RATIO = 3.0
EPS = 1e-06
NEGATIVE_OVERRIDE = 0.1

NC, NS, L = 2, 16, 16
NW = NC * NS
H, W = 1280, 1280
N_PIX = H * W
TR = 8
N_BANDS = H // TR
BANDS_W = N_BANDS // NW
CHUNK = TR * W
UNROLL = 4
NB = 2048
LOSS_HI = 9.25
BIN_SCALE = NB / LOSS_HI

LN2 = 0.6931471805599453
SQRT2 = 1.4142135623730951
_P5 = (-0.14338312557646554, 0.22070299637323865, -0.2539783176623159,
       0.33256777805214044, -0.4999036158820021, 1.0000046848976198)
_P4 = (0.17721477123387214, -0.2711059246189473, 0.33632475570352544,
       -0.49944110881933895, 0.9999670988417516)


def _ln(x, coefs=_P5):
    xi = lax.bitcast_convert_type(x, jnp.int32)
    e = xi >> 23
    m = lax.bitcast_convert_type((xi & 0x007FFFFF) | 0x3F800000, jnp.float32)
    big = m > SQRT2
    m = jnp.where(big, m * 0.5, m)
    ef = (e + jnp.where(big, -126, -127)).astype(jnp.float32)
    r = m - 1.0
    p = coefs[0]
    for c in coefs[1:]:
        p = p * r + c
    return ef * LN2 + r * p


def _losses(p, g, m, ign):
    vm = jnp.where(g == ign, 0.0, m)
    x = jnp.where(g == 1.0, p, 1.0 - p)
    loss = -jnp.maximum(_ln(x), -100.0)
    posf = vm * g
    negf = vm - posf
    return loss, posf, negf


def _sc_stats_body(pred_h, gt_h, mask_h, ign_h, part_h,
                   pred_v, gt_v, mask_v, ign_v, part_v, sem0, sem1):
    wid = lax.axis_index("s") * NC + lax.axis_index("c")
    sems = (sem0, sem1)

    pltpu.sync_copy(ign_h, ign_v)
    zeros = jnp.zeros((L,), jnp.float32)
    ign = ign_v[...]
    acc = ((zeros, zeros, zeros, zeros),) * UNROLL

    def start_band(ci):
        b = ci % 2
        r0 = pl.multiple_of((wid * BANDS_W + ci) * TR, TR)
        return [
            pltpu.async_copy(pred_h.at[pl.ds(r0, TR), :], pred_v.at[b], sems[b]),
            pltpu.async_copy(gt_h.at[pl.ds(r0, TR), :], gt_v.at[b], sems[b]),
            pltpu.async_copy(mask_h.at[pl.ds(r0, TR), :], mask_v.at[b], sems[b]),
        ]

    def make_col_body(b):
        def lane_stats(row, sl, a):
            slv, slvg, svm, svmg = a
            p = pred_v[b, row, sl]
            g = gt_v[b, row, sl]
            m = mask_v[b, row, sl]
            vm = jnp.where(g == ign, 0.0, m)
            x = jnp.where(g == 1.0, p, 1.0 - p)
            lnx = _ln(x, _P4)
            vmg = vm * g
            lv = lnx * vm
            lvg = lnx * vmg
            return (slv + lv, slvg + lvg, svm + vm, svmg + vmg)

        def col_body(j, accs):
            col = pl.multiple_of(j * (L * UNROLL), L * UNROLL)
            accs = list(accs)
            for row in range(TR):
                for u in range(UNROLL):
                    accs[u] = lane_stats(row, pl.ds(col + u * L, L), accs[u])
            return tuple(accs)
        return col_body

    descs = start_band(0)
    for ci in range(BANDS_W):
        nxt = start_band(ci + 1) if ci + 1 < BANDS_W else []
        for d in descs:
            d.wait()
        descs = nxt
        acc = lax.fori_loop(0, W // (L * UNROLL), make_col_body(ci % 2), acc)

    slv, slvg, svm, svmg = acc[0]
    for u in range(1, UNROLL):
        slv = slv + acc[u][0]
        slvg = slvg + acc[u][1]
        svm = svm + acc[u][2]
        svmg = svmg + acc[u][3]
    part_v[0, :] = -slvg
    part_v[1, :] = svmg
    part_v[2, :] = svm - svmg
    part_v[3, :] = slvg - slv
    pltpu.sync_copy(part_v, part_h.at[wid])


def _sc_hist_body(pred_h, gt_h, mask_h, ign_h, hist_h,
                  pred_v, gt_v, mask_v, ign_v, hcnt_v, hsum_v):
    wid = lax.axis_index("s") * NC + lax.axis_index("c")

    pltpu.sync_copy(ign_h, ign_v)
    zeros = jnp.zeros((L,), jnp.float32)
    for j in range(NB // L):
        hcnt_v[j * L:(j + 1) * L] = zeros
        hsum_v[j * L:(j + 1) * L] = zeros

    ign = ign_v[...]
    ones = jnp.ones((L,), jnp.float32)

    def make_col_body(row):
        def col_body(j, carry):
            sl = pl.ds(pl.multiple_of(j * L, L), L)
            loss, _, negf = _losses(
                pred_v[row, sl], gt_v[row, sl], mask_v[row, sl], ign)
            nl = loss * negf
            bi = jnp.minimum((nl * BIN_SCALE).astype(jnp.int32), NB - 1)
            sel = negf > 0.0
            plsc.addupdate_scatter(hcnt_v, [bi], ones, mask=sel)
            plsc.addupdate_scatter(hsum_v, [bi], nl, mask=sel)
            return carry
        return col_body

    for ci in range(BANDS_W):
        r0 = pl.multiple_of((wid * BANDS_W + ci) * TR, TR)
        pltpu.sync_copy(pred_h.at[pl.ds(r0, TR), :], pred_v)
        pltpu.sync_copy(gt_h.at[pl.ds(r0, TR), :], gt_v)
        pltpu.sync_copy(mask_h.at[pl.ds(r0, TR), :], mask_v)
        for row in range(TR):
            lax.fori_loop(0, W // L, make_col_body(row), 0)

    pltpu.sync_copy(hcnt_v, hist_h.at[wid, 0])
    pltpu.sync_copy(hsum_v, hist_h.at[wid, 1])


def kernel(pred, gt, mask, ignore_label=255):
    predf = pred.reshape(H, W)
    gtf = gt.reshape(H, W)
    maskf = mask.reshape(H, W)
    ignf = jnp.full((L,), ignore_label, jnp.float32)

    mesh = plsc.VectorSubcoreMesh(core_axis_name="c", subcore_axis_name="s")
    params = pltpu.CompilerParams(needs_layout_passes=False,
                                  use_tc_tiling_on_sc=True)

    part = pl.kernel(
        _sc_stats_body,
        out_type=jax.ShapeDtypeStruct((NW, 4, L), jnp.float32),
        mesh=mesh,
        compiler_params=params,
        scratch_types=[
            pltpu.VMEM((2, TR, W), jnp.float32),
            pltpu.VMEM((2, TR, W), jnp.float32),
            pltpu.VMEM((2, TR, W), jnp.float32),
            pltpu.VMEM((L,), jnp.float32),
            pltpu.VMEM((4, L), jnp.float32),
            pltpu.SemaphoreType.DMA,
            pltpu.SemaphoreType.DMA,
        ],
    )(predf, gtf, maskf, ignf)

    pos_sum = part[:, 0, :].sum()
    pos_cnt = jnp.floor(part[:, 1, :].sum())
    neg_tot = jnp.floor(part[:, 2, :].sum())
    neg_sum = part[:, 3, :].sum()

    k = jnp.where(
        pos_cnt >= 1.0,
        jnp.minimum(neg_tot, jnp.floor(pos_cnt * NEGATIVE_RATIO)),
        jnp.floor(neg_tot * NEGATIVE_OVERRIDE),
    )

    def topk_from_hist():
        hist = pl.kernel(
            _sc_hist_body,
            out_type=jax.ShapeDtypeStruct((NW, 2, NB), jnp.float32),
            mesh=mesh,
            compiler_params=params,
            scratch_types=[
                pltpu.VMEM((TR, W), jnp.float32),
                pltpu.VMEM((TR, W), jnp.float32),
                pltpu.VMEM((TR, W), jnp.float32),
                pltpu.VMEM((L,), jnp.float32),
                pltpu.VMEM((NB,), jnp.float32),
                pltpu.VMEM((NB,), jnp.float32),
            ],
        )(predf, gtf, maskf, ignf)
        hcnt = hist[:, 0, :].sum(axis=0)
        hsum = hist[:, 1, :].sum(axis=0)
        cc = jnp.cumsum(hcnt[::-1])[::-1]
        cs = jnp.cumsum(hsum[::-1])[::-1]
        cc1 = jnp.concatenate([cc, jnp.zeros((1,), jnp.float32)])
        cs1 = jnp.concatenate([cs, jnp.zeros((1,), jnp.float32)])
        t = jnp.clip((cc >= k).sum() - 1, 0, NB - 1)
        frac = (k - cc1[t + 1]) / jnp.maximum(hcnt[t], 1.0)
        return jnp.where(k >= 1.0, cs1[t + 1] + frac * hsum[t], 0.0)

    neg_top = lax.cond(k >= neg_tot, lambda: neg_sum, topk_from_hist)
    return (pos_sum + neg_top) / (pos_cnt + k + EPS)
